# Initial kernel scaffold; baseline (speedup 1.0000x reference)
import jax
import jax.numpy as jnp
from jax.experimental import pallas as pl


def kernel(x, edge_index, W):
    raise NotImplementedError("write your pallas kernel here")


# speedup vs baseline: 2.5478x; 2.5478x over previous
import jax
import jax.numpy as jnp
from jax.experimental import pallas as pl

N_NODES = 10000
N_EDGES = 320000
D = 128


def _matmul_body(x_ref, w_ref, o_ref):
    o_ref[...] = jnp.dot(x_ref[...], w_ref[...],
                         preferred_element_type=jnp.float32)


def _transform(x, W):
    return pl.pallas_call(
        _matmul_body,
        out_shape=jax.ShapeDtypeStruct((N_NODES, D), jnp.float32),
        grid=(10,),
        in_specs=[
            pl.BlockSpec((N_NODES // 10, D), lambda i: (i, 0)),
            pl.BlockSpec((D, D), lambda i: (0, 0)),
        ],
        out_specs=pl.BlockSpec((N_NODES // 10, D), lambda i: (i, 0)),
    )(x, W)


def kernel(x, edge_index, W):
    src = edge_index[0].astype(jnp.int32)
    dst = edge_index[1].astype(jnp.int32)
    y = _transform(x, W)
    msg = y[src]
    logits = jnp.sum(msg * x[dst], axis=-1)
    e = jnp.exp(logits)
    seg_sum = jax.ops.segment_sum(e, dst, num_segments=N_NODES) + 1e-16
    agg = jax.ops.segment_sum(e[:, None] * msg, dst, num_segments=N_NODES)
    agg = agg / seg_sum[:, None]
    n2 = jnp.sum(agg * agg, axis=-1, keepdims=True)
    scale = n2 / (1.0 + n2) / jnp.sqrt(n2 + 1e-9)
    return agg * scale


# speedup vs baseline: 4.1767x; 1.6394x over previous
import dataclasses
import functools

import jax
import jax.numpy as jnp
from jax import lax
from jax.experimental import pallas as pl
from jax.experimental.pallas import tpu as pltpu
from jax.experimental.pallas import tpu_sc as plsc

N = 10000
NP = 10240
E = 320000
D = 128
L = 16
NC = 2
NS = 16
NW = NC * NS
EPT = E // NW
CHUNK = 80
NCHUNK = EPT // CHUNK
GROUPS = CHUNK // L
RPT = NP // NS


def _matmul_body(x_ref, w_ref, o_ref):
    o_ref[...] = jnp.dot(x_ref[...], w_ref[...],
                         preferred_element_type=jnp.float32)


def _transform(x, W):
    return pl.pallas_call(
        _matmul_body,
        out_shape=jax.ShapeDtypeStruct((N, D), jnp.float32),
        grid=(10,),
        in_specs=[
            pl.BlockSpec((N // 10, D), lambda i: (i, 0)),
            pl.BlockSpec((D, D), lambda i: (0, 0)),
        ],
        out_specs=pl.BlockSpec((N // 10, D), lambda i: (i, 0)),
    )(x, W)


_SC_PARAMS = pltpu.CompilerParams()
if "needs_layout_passes" in pltpu.CompilerParams.__dataclass_fields__:
    _SC_PARAMS = dataclasses.replace(_SC_PARAMS, needs_layout_passes=False)


@functools.partial(
    pl.kernel,
    compiler_params=_SC_PARAMS,
    out_type=(
        jax.ShapeDtypeStruct((NC, NP, D), jnp.float32),
        jax.ShapeDtypeStruct((NC, NP, L), jnp.float32),
    ),
    mesh=plsc.VectorSubcoreMesh(core_axis_name="c", subcore_axis_name="s",
                                num_cores=NC, num_subcores=NS),
    scratch_types=[
        pltpu.VMEM_SHARED((NP, D), jnp.float32),
        pltpu.VMEM_SHARED((NP, L), jnp.float32),
        pltpu.VMEM((CHUNK,), jnp.int32),
        pltpu.VMEM((CHUNK,), jnp.int32),
        pltpu.VMEM((CHUNK,), jnp.int32),
        pltpu.VMEM((CHUNK, D), jnp.float32),
        pltpu.VMEM((CHUNK, D), jnp.float32),
        pltpu.VMEM((CHUNK, L), jnp.float32),
    ],
)
def _edge_kernel(y_hbm, x_hbm, src_hbm, dst_hbm, agg_out, sum_out,
                 agg_sh, sum_sh, src_v, dst_v, win_v, yrows, xrows, estage):
    cid = lax.axis_index("c")
    sid = lax.axis_index("s")
    wid = sid * NC + cid

    zero16 = jnp.zeros((L,), jnp.float32)
    iota16 = lax.iota(jnp.int32, L)
    zero16i = jnp.zeros((L,), jnp.int32)
    row0 = sid * RPT

    @pl.loop(0, CHUNK)
    def _(i):
        for t in range(D // L):
            yrows[i, pl.ds(L * t, L)] = zero16
        estage[i, :] = zero16

    @pl.loop(0, RPT // CHUNK)
    def _(r):
        w0 = row0 + CHUNK * r
        for t in range(CHUNK // L):
            win_v[pl.ds(L * t, L)] = iota16 + (w0 + L * t)
        pltpu.sync_copy(yrows, agg_sh.at[win_v])
        pltpu.sync_copy(estage, sum_sh.at[win_v])
    plsc.subcore_barrier()

    @pl.loop(0, NCHUNK)
    def _(c):
        base = wid * EPT + c * CHUNK
        pltpu.sync_copy(src_hbm.at[pl.ds(base, CHUNK)], src_v)
        pltpu.sync_copy(dst_hbm.at[pl.ds(base, CHUNK)], dst_v)
        pltpu.sync_copy(y_hbm.at[src_v], yrows)
        pltpu.sync_copy(x_hbm.at[dst_v], xrows)
        for g in range(GROUPS):
            rowids = iota16 + (g * L)

            def dot_body(k, acc):
                colk = jnp.full((L,), 0, jnp.int32) + k
                a = plsc.load_gather(yrows, [rowids, colk])
                b = plsc.load_gather(xrows, [rowids, colk])
                return acc + a * b

            acc = lax.fori_loop(0, D, dot_body, zero16, unroll=8)
            e16 = jnp.exp(acc)
            plsc.store_scatter(estage, [rowids, zero16i], e16)
            for j in range(L):
                ej = g * L + j
                eb = plsc.load_gather(
                    estage, [jnp.full((L,), ej, jnp.int32), zero16i])
                for t in range(D // L):
                    sl = pl.ds(L * t, L)
                    yrows[ej, sl] = yrows[ej, sl] * eb
        pltpu.sync_copy(yrows, agg_sh.at[dst_v], add=True)
        pltpu.sync_copy(estage, sum_sh.at[dst_v], add=True)

    plsc.subcore_barrier()

    @pl.loop(0, RPT // CHUNK)
    def _(r):
        w0 = row0 + CHUNK * r
        for t in range(CHUNK // L):
            win_v[pl.ds(L * t, L)] = iota16 + (w0 + L * t)
        pltpu.sync_copy(agg_sh.at[win_v], yrows)
        pltpu.sync_copy(sum_sh.at[win_v], estage)
        pltpu.sync_copy(yrows, agg_out.at[cid, pl.ds(w0, CHUNK)])
        pltpu.sync_copy(estage, sum_out.at[cid, pl.ds(w0, CHUNK)])


def _combine_body(pa_ref, ps_ref, o_ref):
    agg = pa_ref[0] + pa_ref[1]
    s = ps_ref[0, :, 0:1] + ps_ref[1, :, 0:1]
    agg = agg / (s + 1e-16)
    n2 = jnp.sum(agg * agg, axis=-1, keepdims=True)
    o_ref[...] = agg * (n2 / (1.0 + n2) / jnp.sqrt(n2 + 1e-9))


def _combine(pa, ps):
    return pl.pallas_call(
        _combine_body,
        out_shape=jax.ShapeDtypeStruct((NP, D), jnp.float32),
        grid=(5,),
        in_specs=[
            pl.BlockSpec((NC, NP // 5, D), lambda i: (0, i, 0)),
            pl.BlockSpec((NC, NP // 5, L), lambda i: (0, i, 0)),
        ],
        out_specs=pl.BlockSpec((NP // 5, D), lambda i: (i, 0)),
    )(pa, ps)


def kernel(x, edge_index, W):
    ei = edge_index.astype(jnp.int32)
    src = ei[0]
    dst = ei[1]
    y = _transform(x, W)
    pa, ps = _edge_kernel(y, x, src, dst)
    return _combine(pa, ps)[:N]
